# Initial kernel scaffold; baseline (speedup 1.0000x reference)
import jax
import jax.numpy as jnp
from jax.experimental import pallas as pl


def kernel(feats, coords, params, index_0, index_1, index_0_offsets, n_max):
    raise NotImplementedError("write your pallas kernel here")


# speedup vs baseline: 34.6756x; 34.6756x over previous
import jax
import jax.numpy as jnp
from jax import lax
from jax.experimental import pallas as pl
from jax.experimental.pallas import tpu as pltpu
from jax.experimental.pallas import tpu_sc as plsc

N = 10000
K = 16
C = 128
H = 8
HC = C // H
WS = 0.16
QS = 0.04
QGL = int((2 * WS + 1e-4) // QS)
L = 2 * QGL
SCALE = HC ** -0.5
P = N * K


ROWS_A = 1000
GD = 3 * C


def _ln_qkv_body(x_ref, c_ref, s_ref, b_ref, w_ref, bias_ref, qs_ref, kvc_ref):
    x = x_ref[...]
    mu = jnp.mean(x, axis=-1, keepdims=True)
    xc = x - mu
    var = jnp.mean(xc * xc, axis=-1, keepdims=True)
    h = xc / jnp.sqrt(var + 1e-5) * s_ref[...] + b_ref[...]
    qkv = jnp.dot(h, w_ref[...], preferred_element_type=jnp.float32) + bias_ref[...]
    qs_ref[...] = qkv[:, :C] * SCALE
    kvc_ref[:, : 2 * C] = qkv[:, C:]
    kvc_ref[:, 2 * C :] = c_ref[...]


def _ln_qkv(feats, coords128, ln_s, ln_b, w_qkv, b_qkv):
    grid = (N // ROWS_A,)
    return pl.pallas_call(
        _ln_qkv_body,
        grid=grid,
        in_specs=[
            pl.BlockSpec((ROWS_A, C), lambda i: (i, 0)),
            pl.BlockSpec((ROWS_A, C), lambda i: (i, 0)),
            pl.BlockSpec((1, C), lambda i: (0, 0)),
            pl.BlockSpec((1, C), lambda i: (0, 0)),
            pl.BlockSpec((C, 3 * C), lambda i: (0, 0)),
            pl.BlockSpec((1, 3 * C), lambda i: (0, 0)),
        ],
        out_specs=[
            pl.BlockSpec((ROWS_A, C), lambda i: (i, 0)),
            pl.BlockSpec((ROWS_A, GD), lambda i: (i, 0)),
        ],
        out_shape=[
            jax.ShapeDtypeStruct((N, C), jnp.float32),
            jax.ShapeDtypeStruct((N, GD), jnp.float32),
        ],
    )(feats, coords128, ln_s, ln_b, w_qkv, b_qkv)


CHUNK = 128
NC = 2
NS = 16
NW = NC * NS
NCHUNKS = P // CHUNK
ITERS = (NCHUNKS + NW - 1) // NW


def _gather_pairs(kvc, index_1):
    mesh = plsc.VectorSubcoreMesh(
        core_axis_name="c", subcore_axis_name="s", num_cores=NC, num_subcores=NS
    )

    def body(kvc_hbm, idx_hbm, out_hbm, idx_v, rows_v, sem):
        wid = lax.axis_index("s") * NC + lax.axis_index("c")

        def step(j, carry):
            chunk = j * NW + wid

            @pl.when(chunk < NCHUNKS)
            def _():
                base = chunk * CHUNK
                pltpu.sync_copy(idx_hbm.at[pl.ds(base, CHUNK)], idx_v)
                pltpu.async_copy(kvc_hbm.at[idx_v], rows_v, sem).wait()
                pltpu.sync_copy(rows_v, out_hbm.at[pl.ds(base, CHUNK)])

            return carry

        lax.fori_loop(0, ITERS, step, 0)

    f = pl.kernel(
        body,
        out_type=jax.ShapeDtypeStruct((P, GD), jnp.float32),
        mesh=mesh,
        scratch_types=[
            pltpu.VMEM((CHUNK,), jnp.int32),
            pltpu.VMEM((CHUNK, GD), jnp.float32),
            pltpu.SemaphoreType.DMA,
        ],
    )
    return f(kvc, index_1)


TQ = 200
RP = TQ * K


def _attn_body(g_ref, q_ref, c_ref, r_ref, t_ref, w_ref, b_ref, o_ref):
    g = g_ref[...]
    q = q_ref[...]
    kg = g[:, :C]
    vg = g[:, C : 2 * C]
    cg = g[:, 2 * C : 2 * C + 3]

    cq = c_ref[...]
    cqg = jnp.broadcast_to(cq[:, None, :], (TQ, K, 16)).reshape(RP, 16)[:, :3]
    rel = cqg - cg
    rel = jnp.round(rel * 100000.0) / 100000.0
    rpi = jnp.floor((rel + 2 * WS - 1e-4) / QS).astype(jnp.int32)
    rpi = jnp.clip(rpi, 0, L - 1)

    cols = lax.broadcasted_iota(jnp.int32, (RP, 3 * L), 1)
    oh = (
        (cols == rpi[:, 0:1])
        | (cols == rpi[:, 1:2] + L)
        | (cols == rpi[:, 2:3] + 2 * L)
    ).astype(jnp.float32)
    G = jnp.dot(oh, t_ref[...], preferred_element_type=jnp.float32)

    qg = jnp.broadcast_to(q[:, None, :], (TQ, K, C)).reshape(RP, C)
    s = qg * (kg + G[:, :C]) + kg * G[:, C : 2 * C]
    attn = s.reshape(RP, H, HC).sum(axis=-1)

    a3 = attn.reshape(TQ, K, H)
    m = jnp.max(a3, axis=1, keepdims=True)
    e = jnp.exp(a3 - m)
    den = jnp.sum(e, axis=1, keepdims=True)
    p = (e / den).reshape(RP, H)

    vt = vg + G[:, 2 * C :]
    pb = jnp.broadcast_to(p[:, :, None], (RP, H, HC)).reshape(RP, C)
    o = (pb * vt).reshape(TQ, K, C).sum(axis=1)

    x = jnp.dot(o, w_ref[...], preferred_element_type=jnp.float32)
    o_ref[...] = x + b_ref[...] + r_ref[...]


def _attention(g, qs, coords16, resid, t_stack, w_proj, b_proj):
    grid = (N // TQ,)
    return pl.pallas_call(
        _attn_body,
        grid=grid,
        in_specs=[
            pl.BlockSpec((RP, GD), lambda i: (i, 0)),
            pl.BlockSpec((TQ, C), lambda i: (i, 0)),
            pl.BlockSpec((TQ, 16), lambda i: (i, 0)),
            pl.BlockSpec((TQ, C), lambda i: (i, 0)),
            pl.BlockSpec((3 * L, 3 * C), lambda i: (0, 0)),
            pl.BlockSpec((C, C), lambda i: (0, 0)),
            pl.BlockSpec((1, C), lambda i: (0, 0)),
        ],
        out_specs=pl.BlockSpec((TQ, C), lambda i: (i, 0)),
        out_shape=jax.ShapeDtypeStruct((N, C), jnp.float32),
    )(g, qs, coords16, resid, t_stack, w_proj, b_proj)


ROWS_D = 1000
HID = 4 * C


def _mlp_body(x_ref, s_ref, b_ref, w1_ref, b1_ref, w2_ref, b2_ref, o_ref):
    x = x_ref[...]
    mu = jnp.mean(x, axis=-1, keepdims=True)
    xc = x - mu
    var = jnp.mean(xc * xc, axis=-1, keepdims=True)
    h = xc / jnp.sqrt(var + 1e-5) * s_ref[...] + b_ref[...]
    f = jax.nn.gelu(jnp.dot(h, w1_ref[...], preferred_element_type=jnp.float32) + b1_ref[...])
    o_ref[...] = x + jnp.dot(f, w2_ref[...], preferred_element_type=jnp.float32) + b2_ref[...]


def _mlp(x, ln_s, ln_b, w1, b1, w2, b2):
    grid = (N // ROWS_D,)
    return pl.pallas_call(
        _mlp_body,
        grid=grid,
        in_specs=[
            pl.BlockSpec((ROWS_D, C), lambda i: (i, 0)),
            pl.BlockSpec((1, C), lambda i: (0, 0)),
            pl.BlockSpec((1, C), lambda i: (0, 0)),
            pl.BlockSpec((C, HID), lambda i: (0, 0)),
            pl.BlockSpec((1, HID), lambda i: (0, 0)),
            pl.BlockSpec((HID, C), lambda i: (0, 0)),
            pl.BlockSpec((1, C), lambda i: (0, 0)),
        ],
        out_specs=pl.BlockSpec((ROWS_D, C), lambda i: (i, 0)),
        out_shape=jax.ShapeDtypeStruct((N, C), jnp.float32),
    )(x, ln_s, ln_b, w1, b1, w2, b2)


def _stack_tables(p):
    parts = []
    for name in ("tab_q", "tab_k", "tab_v"):
        t = p[name]
        parts.append(jnp.concatenate([t[:, :, :, d].reshape(L, C) for d in range(3)], axis=0))
    return jnp.concatenate(parts, axis=1)


def kernel(feats, coords, params, index_0, index_1, index_0_offsets, n_max):
    coords16 = jnp.pad(coords, ((0, 0), (0, 13)))
    coords128 = jnp.pad(coords, ((0, 0), (0, C - 3)))
    x = feats
    for p in params["blocks"]:
        t_stack = _stack_tables(p)
        qs, kvc = _ln_qkv(
            x,
            coords128,
            p["ln1_s"].reshape(1, C),
            p["ln1_b"].reshape(1, C),
            p["w_qkv"],
            p["b_qkv"].reshape(1, 3 * C),
        )
        g = _gather_pairs(kvc, index_1)
        x = _attention(g, qs, coords16, x, t_stack, p["w_proj"], p["b_proj"].reshape(1, C))
        x = _mlp(
            x,
            p["ln2_s"].reshape(1, C),
            p["ln2_b"].reshape(1, C),
            p["w_fc1"],
            p["b_fc1"].reshape(1, HID),
            p["w_fc2"],
            p["b_fc2"].reshape(1, C),
        )
    return x
